# baseline (speedup 1.0000x reference)
import functools

import jax
import jax.numpy as jnp
from jax import lax
from jax.experimental import pallas as pl
from jax.experimental.pallas import tpu as pltpu
from jax.experimental.pallas import tpu_sc as plsc


def _embed_body(nodes_ref, we_ref, be_ref, out_ref):
    out_ref[...] = (
        jnp.dot(nodes_ref[...], we_ref[...], preferred_element_type=jnp.float32)
        + be_ref[...]
    )


def _mlp_of(x, w1_ref, b1_ref, w2_ref, b2_ref, degs_ref):
    h = jnp.dot(x, w1_ref[...], preferred_element_type=jnp.float32)
    h = jnp.maximum(h + b1_ref[...], 0.0)
    h = jnp.dot(h, w2_ref[...], preferred_element_type=jnp.float32) + b2_ref[...]
    d = degs_ref[0] + degs_ref[1]
    return h * lax.rsqrt(d[:, 0:1] + 1.0)


def _embed_mlp_body(nodes_ref, we_ref, be_ref, w1_ref, b1_ref, w2_ref, b2_ref,
                    degs_ref, x_ref, hs_ref):
    x = (jnp.dot(nodes_ref[...], we_ref[...], preferred_element_type=jnp.float32)
         + be_ref[...])
    x_ref[...] = x
    hs_ref[...] = _mlp_of(x, w1_ref, b1_ref, w2_ref, b2_ref, degs_ref)


def _update_of(x_ref, p_ref, hs_ref, degr_ref, lns_ref, lnb_ref):
    msg = p_ref[0] + p_ref[1] + hs_ref[...]
    d = degr_ref[0] + degr_ref[1]
    inv = lax.rsqrt(d[:, 0:1] + 1.0)
    x = x_ref[...] + msg * inv
    mu = jnp.mean(x, axis=-1, keepdims=True)
    var = jnp.mean(jnp.square(x - mu), axis=-1, keepdims=True)
    return (x - mu) * lax.rsqrt(var + 1e-6) * lns_ref[...] + lnb_ref[...]


def _update_mlp_body(x_ref, p_ref, hs_ref, degr_ref, lns_ref, lnb_ref,
                     w1_ref, b1_ref, w2_ref, b2_ref, degs_ref,
                     xo_ref, hso_ref):
    x = _update_of(x_ref, p_ref, hs_ref, degr_ref, lns_ref, lnb_ref)
    xo_ref[...] = x
    hso_ref[...] = _mlp_of(x, w1_ref, b1_ref, w2_ref, b2_ref, degs_ref)


def _update_pool_body(x_ref, p_ref, hs_ref, degr_ref, lns_ref, lnb_ref,
                      seg_ref, cnt_ref, wd_ref, bd_ref, out_ref, acc_ref):
    i = pl.program_id(0)
    x = _update_of(x_ref, p_ref, hs_ref, degr_ref, lns_ref, lnb_ref)
    part = lax.dot_general(seg_ref[...], x, (((0,), (0,)), ((), ())),
                           preferred_element_type=jnp.float32)

    @pl.when(i == 0)
    def _():
        acc_ref[...] = part

    @pl.when(i > 0)
    def _():
        acc_ref[...] += part

    @pl.when(i == pl.num_programs(0) - 1)
    def _():
        pooled = acc_ref[...] / cnt_ref[...]
        out_ref[...] = (
            jnp.dot(pooled, wd_ref[...], preferred_element_type=jnp.float32)
            + bd_ref[...]
        )


def _mlp_body(x_ref, w1_ref, b1_ref, w2_ref, b2_ref, degs_ref, out_ref):
    h = jnp.dot(x_ref[...], w1_ref[...], preferred_element_type=jnp.float32)
    h = jnp.maximum(h + b1_ref[...], 0.0)
    h = jnp.dot(h, w2_ref[...], preferred_element_type=jnp.float32) + b2_ref[...]
    d = degs_ref[0] + degs_ref[1]
    inv = lax.rsqrt(d[:, 0:1] + 1.0)
    out_ref[...] = h * inv


def _update_body(x_ref, p_ref, hs_ref, degr_ref, lns_ref, lnb_ref, out_ref):
    msg = p_ref[0] + p_ref[1] + hs_ref[...]
    d = degr_ref[0] + degr_ref[1]
    inv = lax.rsqrt(d[:, 0:1] + 1.0)
    x = x_ref[...] + msg * inv
    mu = jnp.mean(x, axis=-1, keepdims=True)
    var = jnp.mean(jnp.square(x - mu), axis=-1, keepdims=True)
    out_ref[...] = (x - mu) * lax.rsqrt(var + 1e-6) * lns_ref[...] + lnb_ref[...]


def _pool_body(x_ref, seg_ref, cnt_ref, wd_ref, bd_ref, out_ref):
    pooled = jnp.dot(seg_ref[...], x_ref[...], preferred_element_type=jnp.float32)
    pooled = pooled / cnt_ref[...]
    out_ref[...] = (
        jnp.dot(pooled, wd_ref[...], preferred_element_type=jnp.float32)
        + bd_ref[...]
    )


def _row_grid_specs(N, R, D, extra_in):
    grid = (N // R,)
    in_specs = [pl.BlockSpec((R, D), lambda i: (i, 0))] + extra_in
    out_spec = pl.BlockSpec((R, D), lambda i: (i, 0))
    return grid, in_specs, out_spec


def _tc_embed(nodes, W_embed, b_embed, R):
    N, D = nodes.shape
    L = W_embed.shape[1]
    grid, in_specs, out_spec = _row_grid_specs(N, R, D, [
        pl.BlockSpec((D, L), lambda i: (0, 0)),
        pl.BlockSpec((1, L), lambda i: (0, 0)),
    ])
    return pl.pallas_call(
        _embed_body,
        grid=grid,
        in_specs=in_specs,
        out_specs=out_spec,
        out_shape=jax.ShapeDtypeStruct((N, L), jnp.float32),
    )(nodes, W_embed, b_embed)


def _tc_mlp(x, W1, b1, W2, b2, deg_s, R):
    N, L = x.shape
    W = deg_s.shape[-1]
    grid, in_specs, out_spec = _row_grid_specs(N, R, L, [
        pl.BlockSpec((L, L), lambda i: (0, 0)),
        pl.BlockSpec((1, L), lambda i: (0, 0)),
        pl.BlockSpec((L, L), lambda i: (0, 0)),
        pl.BlockSpec((1, L), lambda i: (0, 0)),
        pl.BlockSpec((2, R, W), lambda i: (0, i, 0)),
    ])
    return pl.pallas_call(
        _mlp_body,
        grid=grid,
        in_specs=in_specs,
        out_specs=out_spec,
        out_shape=jax.ShapeDtypeStruct((N, L), jnp.float32),
    )(x, W1, b1, W2, b2, deg_s)


def _tc_update(x, p, hs, deg_r, ln_scale, ln_bias, R):
    N, L = x.shape
    W = deg_r.shape[-1]
    grid, in_specs, out_spec = _row_grid_specs(N, R, L, [
        pl.BlockSpec((2, R, L), lambda i: (0, i, 0)),
        pl.BlockSpec((R, L), lambda i: (i, 0)),
        pl.BlockSpec((2, R, W), lambda i: (0, i, 0)),
        pl.BlockSpec((1, L), lambda i: (0, 0)),
        pl.BlockSpec((1, L), lambda i: (0, 0)),
    ])
    return pl.pallas_call(
        _update_body,
        grid=grid,
        in_specs=in_specs,
        out_specs=out_spec,
        out_shape=jax.ShapeDtypeStruct((N, L), jnp.float32),
    )(x, p, hs, deg_r, ln_scale, ln_bias)


def _tc_embed_mlp(nodes, W_embed, b_embed, W1, b1, W2, b2, deg_s, R):
    N, D = nodes.shape
    L = W_embed.shape[1]
    W = deg_s.shape[-1]
    grid = (N // R,)
    specs = [
        pl.BlockSpec((R, D), lambda i: (i, 0)),
        pl.BlockSpec((D, L), lambda i: (0, 0)),
        pl.BlockSpec((1, L), lambda i: (0, 0)),
        pl.BlockSpec((L, L), lambda i: (0, 0)),
        pl.BlockSpec((1, L), lambda i: (0, 0)),
        pl.BlockSpec((L, L), lambda i: (0, 0)),
        pl.BlockSpec((1, L), lambda i: (0, 0)),
        pl.BlockSpec((2, R, W), lambda i: (0, i, 0)),
    ]
    return pl.pallas_call(
        _embed_mlp_body,
        grid=grid,
        in_specs=specs,
        out_specs=[pl.BlockSpec((R, L), lambda i: (i, 0))] * 2,
        out_shape=[jax.ShapeDtypeStruct((N, L), jnp.float32)] * 2,
    )(nodes, W_embed, b_embed, W1, b1, W2, b2, deg_s)


def _tc_update_mlp(x, p, hs, deg_r, ln_scale, ln_bias, W1, b1, W2, b2, deg_s, R):
    N, L = x.shape
    W = deg_r.shape[-1]
    grid = (N // R,)
    specs = [
        pl.BlockSpec((R, L), lambda i: (i, 0)),
        pl.BlockSpec((2, R, L), lambda i: (0, i, 0)),
        pl.BlockSpec((R, L), lambda i: (i, 0)),
        pl.BlockSpec((2, R, W), lambda i: (0, i, 0)),
        pl.BlockSpec((1, L), lambda i: (0, 0)),
        pl.BlockSpec((1, L), lambda i: (0, 0)),
        pl.BlockSpec((L, L), lambda i: (0, 0)),
        pl.BlockSpec((1, L), lambda i: (0, 0)),
        pl.BlockSpec((L, L), lambda i: (0, 0)),
        pl.BlockSpec((1, L), lambda i: (0, 0)),
        pl.BlockSpec((2, R, W), lambda i: (0, i, 0)),
    ]
    return pl.pallas_call(
        _update_mlp_body,
        grid=grid,
        in_specs=specs,
        out_specs=[pl.BlockSpec((R, L), lambda i: (i, 0))] * 2,
        out_shape=[jax.ShapeDtypeStruct((N, L), jnp.float32)] * 2,
    )(x, p, hs, deg_r, ln_scale, ln_bias, W1, b1, W2, b2, deg_s)


def _tc_update_pool(x, p, hs, deg_r, ln_scale, ln_bias, seg, cnt, W_dec, b_dec, R):
    N, L = x.shape
    W = deg_r.shape[-1]
    G = seg.shape[1]
    OG = W_dec.shape[1]
    grid = (N // R,)
    specs = [
        pl.BlockSpec((R, L), lambda i: (i, 0)),
        pl.BlockSpec((2, R, L), lambda i: (0, i, 0)),
        pl.BlockSpec((R, L), lambda i: (i, 0)),
        pl.BlockSpec((2, R, W), lambda i: (0, i, 0)),
        pl.BlockSpec((1, L), lambda i: (0, 0)),
        pl.BlockSpec((1, L), lambda i: (0, 0)),
        pl.BlockSpec((R, G), lambda i: (i, 0)),
        pl.BlockSpec((G, 1), lambda i: (0, 0)),
        pl.BlockSpec((L, OG), lambda i: (0, 0)),
        pl.BlockSpec((1, OG), lambda i: (0, 0)),
    ]
    return pl.pallas_call(
        _update_pool_body,
        grid=grid,
        in_specs=specs,
        out_specs=pl.BlockSpec((G, OG), lambda i: (0, 0)),
        out_shape=jax.ShapeDtypeStruct((G, OG), jnp.float32),
        scratch_shapes=[pltpu.VMEM((G, L), jnp.float32)],
    )(x, p, hs, deg_r, ln_scale, ln_bias, seg, cnt, W_dec, b_dec)


def _tc_pool(x, seg, cnt, W_dec, b_dec):
    N, L = x.shape
    G = seg.shape[0]
    OG = W_dec.shape[1]
    return pl.pallas_call(
        _pool_body,
        in_specs=[
            pl.BlockSpec((N, L), lambda: (0, 0)),
            pl.BlockSpec((G, N), lambda: (0, 0)),
            pl.BlockSpec((G, 1), lambda: (0, 0)),
            pl.BlockSpec((L, OG), lambda: (0, 0)),
            pl.BlockSpec((1, OG), lambda: (0, 0)),
        ],
        out_specs=pl.BlockSpec((G, OG), lambda: (0, 0)),
        out_shape=jax.ShapeDtypeStruct((G, OG), jnp.float32),
    )(x, seg, cnt, W_dec, b_dec)


_DEGW = 8


_C = 128


def _edge_layout(E):
    info = plsc.get_sparse_core_info()
    NW = info.num_cores * info.num_subcores
    per = -(-E // (NW * _C)) * _C
    return NW, per // _C + 2


def _build_edge_chunks(idx, padval, NW, NCHT):
    E = idx.shape[0]
    per_real = (NCHT - 2) * _C
    pad = NW * per_real - E
    full = jnp.concatenate([idx, jnp.full((pad,), padval, idx.dtype)])
    x3 = full.reshape(NW, NCHT - 2, _C)
    dum = jnp.full((NW, 2, _C), padval, idx.dtype)
    return jnp.concatenate([x3, dum], axis=1).reshape(NW * NCHT, _C)


@functools.lru_cache(maxsize=None)
def _make_deg(N, E):
    info = plsc.get_sparse_core_info()
    NC, NS = info.num_cores, info.num_subcores
    NW, NCH = _edge_layout(E)
    C = _C
    NPAD = N + 8
    R0 = (N // NS) & ~7
    rem = N - NS * R0

    mesh = plsc.VectorSubcoreMesh(core_axis_name="c", subcore_axis_name="s")

    @functools.partial(
        pl.kernel,
        mesh=mesh,
        compiler_params=pltpu.CompilerParams(use_tc_tiling_on_sc=False),
        out_type=(
            jax.ShapeDtypeStruct((NC * N, _DEGW), jnp.float32),
            jax.ShapeDtypeStruct((NC * N, _DEGW), jnp.float32),
        ),
        scratch_types=[
            pltpu.VMEM_SHARED((NPAD, _DEGW), jnp.float32),
            pltpu.VMEM_SHARED((NPAD, _DEGW), jnp.float32),
            pltpu.VMEM((NCH, C), jnp.int32),
            pltpu.VMEM((NCH, C), jnp.int32),
            pltpu.VMEM((C, _DEGW), jnp.float32),
            pltpu.SemaphoreType.DMA,
        ],
    )
    def deg(send_hbm, recv_hbm, zeros_hbm, ones_hbm, outs_hbm, outr_hbm,
            accs_sh, accr_sh, sidx_all, ridx_all, ones_v, sem):
        c = lax.axis_index("c")
        s = lax.axis_index("s")
        wid = c * NS + s
        pltpu.sync_copy(zeros_hbm.at[pl.ds(s * R0, R0)],
                        accs_sh.at[pl.ds(s * R0, R0)])
        pltpu.sync_copy(zeros_hbm.at[pl.ds(s * R0, R0)],
                        accr_sh.at[pl.ds(s * R0, R0)])
        if rem:
            @pl.when(s == NS - 1)
            def _():
                pltpu.sync_copy(zeros_hbm.at[pl.ds(NS * R0, rem)],
                                accs_sh.at[pl.ds(NS * R0, rem)])
                pltpu.sync_copy(zeros_hbm.at[pl.ds(NS * R0, rem)],
                                accr_sh.at[pl.ds(NS * R0, rem)])
        pltpu.sync_copy(send_hbm.at[pl.ds(wid * NCH, NCH)], sidx_all)
        pltpu.sync_copy(recv_hbm.at[pl.ds(wid * NCH, NCH)], ridx_all)
        pltpu.sync_copy(ones_hbm, ones_v)
        plsc.subcore_barrier()

        def fire(j, carry):
            pltpu.async_copy(ones_v, accs_sh.at[sidx_all.at[j]], sem, add=True)
            pltpu.async_copy(ones_v, accr_sh.at[ridx_all.at[j]], sem, add=True)
            return carry

        lax.fori_loop(0, NCH, fire, 0)

        def drain(j, carry):
            pltpu.make_async_copy(ones_v, accs_sh.at[sidx_all.at[j]], sem).wait()
            pltpu.make_async_copy(ones_v, accr_sh.at[ridx_all.at[j]], sem).wait()
            return carry

        lax.fori_loop(0, NCH, drain, 0)
        plsc.subcore_barrier()
        pltpu.sync_copy(accs_sh.at[pl.ds(s * R0, R0)],
                        outs_hbm.at[pl.ds(c * N + s * R0, R0)])
        pltpu.sync_copy(accr_sh.at[pl.ds(s * R0, R0)],
                        outr_hbm.at[pl.ds(c * N + s * R0, R0)])
        if rem:
            @pl.when(s == NS - 1)
            def _():
                pltpu.sync_copy(accs_sh.at[pl.ds(NS * R0, rem)],
                                outs_hbm.at[pl.ds(c * N + NS * R0, rem)])
                pltpu.sync_copy(accr_sh.at[pl.ds(NS * R0, rem)],
                                outr_hbm.at[pl.ds(c * N + NS * R0, rem)])

    return deg


@functools.lru_cache(maxsize=None)
def _make_conv(N, D, E):
    info = plsc.get_sparse_core_info()
    NC, NS = info.num_cores, info.num_subcores
    NW, NCHT = _edge_layout(E)
    C = _C
    NPAD = N + 8
    R0 = (N // NS) & ~7
    rem = N - NS * R0

    mesh = plsc.VectorSubcoreMesh(core_axis_name="c", subcore_axis_name="s")

    @functools.partial(
        pl.kernel,
        mesh=mesh,
        compiler_params=pltpu.CompilerParams(use_tc_tiling_on_sc=False),
        out_type=jax.ShapeDtypeStruct((NC * N, D), jnp.float32),
        scratch_types=[
            pltpu.VMEM_SHARED((NPAD, D), jnp.float32),
            pltpu.VMEM((NCHT, C), jnp.int32),
            pltpu.VMEM((C,), jnp.int32),
            pltpu.VMEM((C,), jnp.int32),
            pltpu.VMEM((C, D), jnp.float32),
            pltpu.VMEM((C, D), jnp.float32),
            pltpu.SemaphoreType.DMA,
            pltpu.SemaphoreType.DMA,
        ],
    )
    def conv(hs_hbm, send_hbm, recv_hbm, zeros_hbm, out_hbm,
             acc_sh, sidx_all, ridxA, ridxB, rows0, rows1, g0, g1):
        c = lax.axis_index("c")
        s = lax.axis_index("s")
        wid = c * NS + s
        pltpu.sync_copy(zeros_hbm.at[pl.ds(s * R0, R0)],
                        acc_sh.at[pl.ds(s * R0, R0)])
        if rem:
            @pl.when(s == NS - 1)
            def _():
                pltpu.sync_copy(zeros_hbm.at[pl.ds(NS * R0, rem)],
                                acc_sh.at[pl.ds(NS * R0, rem)])
        pltpu.sync_copy(send_hbm.at[pl.ds(wid * NCHT, NCHT)], sidx_all)
        plsc.subcore_barrier()
        rbase = wid * NCHT

        pltpu.sync_copy(recv_hbm.at[rbase], ridxA)
        pltpu.async_copy(hs_hbm.at[sidx_all.at[0]], rows0, g0)

        def pair(g, carry):
            j = 2 * g
            pltpu.make_async_copy(hs_hbm.at[sidx_all.at[j]], rows0, g0).wait()
            pltpu.sync_copy(recv_hbm.at[rbase + j + 1], ridxB)
            pltpu.async_copy(hs_hbm.at[sidx_all.at[j + 1]], rows1, g1)
            pltpu.sync_copy(rows0, acc_sh.at[ridxA], add=True)
            pltpu.make_async_copy(hs_hbm.at[sidx_all.at[j + 1]], rows1, g1).wait()
            pltpu.sync_copy(recv_hbm.at[rbase + j + 2], ridxA)
            pltpu.async_copy(hs_hbm.at[sidx_all.at[j + 2]], rows0, g0)
            pltpu.sync_copy(rows1, acc_sh.at[ridxB], add=True)
            return carry

        lax.fori_loop(0, (NCHT - 2) // 2, pair, 0)
        pltpu.make_async_copy(hs_hbm.at[sidx_all.at[NCHT - 2]], rows0, g0).wait()
        pltpu.sync_copy(rows0, acc_sh.at[ridxA], add=True)
        plsc.subcore_barrier()
        pltpu.sync_copy(acc_sh.at[pl.ds(s * R0, R0)],
                        out_hbm.at[pl.ds(c * N + s * R0, R0)])
        if rem:
            @pl.when(s == NS - 1)
            def _():
                pltpu.sync_copy(acc_sh.at[pl.ds(NS * R0, rem)],
                                out_hbm.at[pl.ds(c * N + NS * R0, rem)])

    return conv


_STEPS = 2
_ROWS = 2000


def kernel(nodes, senders, receivers, n_node, W_embed, b_embed, W1, b1, W2, b2,
           ln_scale, ln_bias, W_dec, b_dec):
    N, D = nodes.shape
    L = W_embed.shape[1]
    E = senders.shape[0]
    G = n_node.shape[0]
    OG = W_dec.shape[1]

    b_embed2 = b_embed.reshape(1, L)
    b12 = b1.reshape(1, L)
    b22 = b2.reshape(1, L)
    lns2 = ln_scale.reshape(1, L)
    lnb2 = ln_bias.reshape(1, L)
    b_dec2 = b_dec.reshape(1, OG)

    NW, NCHT = _edge_layout(E)
    send_conv = _build_edge_chunks(senders, 0, NW, NCHT)
    send_deg = _build_edge_chunks(senders, N, NW, NCHT)
    recv_pad = _build_edge_chunks(receivers, N, NW, NCHT)

    zeros_deg = jnp.zeros((N, _DEGW), jnp.float32)
    ones_w = jnp.ones((_C, _DEGW), jnp.float32)
    deg_fn = _make_deg(N, E)
    deg_s, deg_r = deg_fn(send_deg, recv_pad, zeros_deg, ones_w)
    info = plsc.get_sparse_core_info()
    NC = info.num_cores
    deg_s = deg_s.reshape(NC, N, _DEGW)
    deg_r = deg_r.reshape(NC, N, _DEGW)

    zeros_nd = jnp.zeros((N, D), jnp.float32)
    conv_fn = _make_conv(N, D, E)

    graph_idx = jnp.repeat(jnp.arange(G, dtype=jnp.int32), n_node, axis=0,
                           total_repeat_length=N)
    seg = (graph_idx[:, None] == jnp.arange(G, dtype=jnp.int32)[None, :]
           ).astype(jnp.float32)
    cnt = jnp.maximum(n_node.astype(jnp.float32), 1.0).reshape(G, 1)

    x, hs = _tc_embed_mlp(nodes, W_embed, b_embed2, W1, b12, W2, b22,
                          deg_s, _ROWS)
    p = conv_fn(hs, send_conv, recv_pad, zeros_nd).reshape(NC, N, D)
    x, hs = _tc_update_mlp(x, p, hs, deg_r, lns2, lnb2,
                           W1, b12, W2, b22, deg_s, _ROWS)
    p = conv_fn(hs, send_conv, recv_pad, zeros_nd).reshape(NC, N, D)
    return _tc_update_pool(x, p, hs, deg_r, lns2, lnb2, seg, cnt,
                           W_dec, b_dec2, _ROWS)


# speedup vs baseline: 1.2610x; 1.2610x over previous
import functools

import jax
import jax.numpy as jnp
from jax import lax
from jax.experimental import pallas as pl
from jax.experimental.pallas import tpu as pltpu
from jax.experimental.pallas import tpu_sc as plsc


def _embed_body(nodes_ref, we_ref, be_ref, out_ref):
    out_ref[...] = (
        jnp.dot(nodes_ref[...], we_ref[...], preferred_element_type=jnp.float32)
        + be_ref[...]
    )


def _mlp_body(x_ref, w1_ref, b1_ref, w2_ref, b2_ref, degs_ref, out_ref):
    h = jnp.dot(x_ref[...], w1_ref[...], preferred_element_type=jnp.float32)
    h = jnp.maximum(h + b1_ref[...], 0.0)
    h = jnp.dot(h, w2_ref[...], preferred_element_type=jnp.float32) + b2_ref[...]
    d = degs_ref[0] + degs_ref[1]
    inv = lax.rsqrt(d[:, 0:1] + 1.0)
    out_ref[...] = h * inv


def _update_body(x_ref, p_ref, hs_ref, degr_ref, lns_ref, lnb_ref, out_ref):
    msg = p_ref[0] + p_ref[1] + hs_ref[...]
    d = degr_ref[0] + degr_ref[1]
    inv = lax.rsqrt(d[:, 0:1] + 1.0)
    x = x_ref[...] + msg * inv
    mu = jnp.mean(x, axis=-1, keepdims=True)
    var = jnp.mean(jnp.square(x - mu), axis=-1, keepdims=True)
    out_ref[...] = (x - mu) * lax.rsqrt(var + 1e-6) * lns_ref[...] + lnb_ref[...]


def _pool_body(x_ref, seg_ref, cnt_ref, wd_ref, bd_ref, out_ref):
    pooled = jnp.dot(seg_ref[...], x_ref[...], preferred_element_type=jnp.float32)
    pooled = pooled / cnt_ref[...]
    out_ref[...] = (
        jnp.dot(pooled, wd_ref[...], preferred_element_type=jnp.float32)
        + bd_ref[...]
    )


def _row_grid_specs(N, R, D, extra_in):
    grid = (N // R,)
    in_specs = [pl.BlockSpec((R, D), lambda i: (i, 0))] + extra_in
    out_spec = pl.BlockSpec((R, D), lambda i: (i, 0))
    return grid, in_specs, out_spec


def _tc_embed(nodes, W_embed, b_embed, R):
    N, D = nodes.shape
    L = W_embed.shape[1]
    grid, in_specs, out_spec = _row_grid_specs(N, R, D, [
        pl.BlockSpec((D, L), lambda i: (0, 0)),
        pl.BlockSpec((1, L), lambda i: (0, 0)),
    ])
    return pl.pallas_call(
        _embed_body,
        grid=grid,
        in_specs=in_specs,
        out_specs=out_spec,
        out_shape=jax.ShapeDtypeStruct((N, L), jnp.float32),
    )(nodes, W_embed, b_embed)


def _tc_mlp(x, W1, b1, W2, b2, deg_s, R):
    N, L = x.shape
    W = deg_s.shape[-1]
    grid, in_specs, out_spec = _row_grid_specs(N, R, L, [
        pl.BlockSpec((L, L), lambda i: (0, 0)),
        pl.BlockSpec((1, L), lambda i: (0, 0)),
        pl.BlockSpec((L, L), lambda i: (0, 0)),
        pl.BlockSpec((1, L), lambda i: (0, 0)),
        pl.BlockSpec((2, R, W), lambda i: (0, i, 0)),
    ])
    return pl.pallas_call(
        _mlp_body,
        grid=grid,
        in_specs=in_specs,
        out_specs=out_spec,
        out_shape=jax.ShapeDtypeStruct((N, L), jnp.float32),
    )(x, W1, b1, W2, b2, deg_s)


def _tc_update(x, p, hs, deg_r, ln_scale, ln_bias, R):
    N, L = x.shape
    W = deg_r.shape[-1]
    grid, in_specs, out_spec = _row_grid_specs(N, R, L, [
        pl.BlockSpec((2, R, L), lambda i: (0, i, 0)),
        pl.BlockSpec((R, L), lambda i: (i, 0)),
        pl.BlockSpec((2, R, W), lambda i: (0, i, 0)),
        pl.BlockSpec((1, L), lambda i: (0, 0)),
        pl.BlockSpec((1, L), lambda i: (0, 0)),
    ])
    return pl.pallas_call(
        _update_body,
        grid=grid,
        in_specs=in_specs,
        out_specs=out_spec,
        out_shape=jax.ShapeDtypeStruct((N, L), jnp.float32),
    )(x, p, hs, deg_r, ln_scale, ln_bias)


def _tc_pool(x, seg, cnt, W_dec, b_dec):
    N, L = x.shape
    G = seg.shape[0]
    OG = W_dec.shape[1]
    return pl.pallas_call(
        _pool_body,
        in_specs=[
            pl.BlockSpec((N, L), lambda: (0, 0)),
            pl.BlockSpec((G, N), lambda: (0, 0)),
            pl.BlockSpec((G, 1), lambda: (0, 0)),
            pl.BlockSpec((L, OG), lambda: (0, 0)),
            pl.BlockSpec((1, OG), lambda: (0, 0)),
        ],
        out_specs=pl.BlockSpec((G, OG), lambda: (0, 0)),
        out_shape=jax.ShapeDtypeStruct((G, OG), jnp.float32),
    )(x, seg, cnt, W_dec, b_dec)


_DEGW = 8
_C = 128


def _edge_layout(E):
    info = plsc.get_sparse_core_info()
    NW = info.num_cores * info.num_subcores
    per = -(-E // (NW * _C)) * _C
    return NW, per // _C + 2


def _build_edge_chunks(idx, padval, NW, NCHT):
    E = idx.shape[0]
    per_real = (NCHT - 2) * _C
    pad = NW * per_real - E
    full = jnp.concatenate([idx, jnp.full((pad,), padval, idx.dtype)])
    x3 = full.reshape(NW, NCHT - 2, _C)
    dum = jnp.full((NW, 2, _C), padval, idx.dtype)
    return jnp.concatenate([x3, dum], axis=1).reshape(NW * NCHT, _C)


@functools.lru_cache(maxsize=None)
def _make_deg(N, E):
    info = plsc.get_sparse_core_info()
    NC, NS = info.num_cores, info.num_subcores
    NW, NCH = _edge_layout(E)
    C = _C
    NPAD = N + 8
    R0 = (N // NS) & ~7
    rem = N - NS * R0

    mesh = plsc.VectorSubcoreMesh(core_axis_name="c", subcore_axis_name="s")

    @functools.partial(
        pl.kernel,
        mesh=mesh,
        compiler_params=pltpu.CompilerParams(use_tc_tiling_on_sc=False),
        out_type=(
            jax.ShapeDtypeStruct((NC * N, _DEGW), jnp.float32),
            jax.ShapeDtypeStruct((NC * N, _DEGW), jnp.float32),
        ),
        scratch_types=[
            pltpu.VMEM_SHARED((NPAD, _DEGW), jnp.float32),
            pltpu.VMEM_SHARED((NPAD, _DEGW), jnp.float32),
            pltpu.VMEM((NCH, C), jnp.int32),
            pltpu.VMEM((NCH, C), jnp.int32),
            pltpu.VMEM((C, _DEGW), jnp.float32),
            pltpu.SemaphoreType.DMA,
        ],
    )
    def deg(send_hbm, recv_hbm, zeros_hbm, ones_hbm, outs_hbm, outr_hbm,
            accs_sh, accr_sh, sidx_all, ridx_all, ones_v, sem):
        c = lax.axis_index("c")
        s = lax.axis_index("s")
        wid = c * NS + s
        pltpu.sync_copy(zeros_hbm.at[pl.ds(s * R0, R0)],
                        accs_sh.at[pl.ds(s * R0, R0)])
        pltpu.sync_copy(zeros_hbm.at[pl.ds(s * R0, R0)],
                        accr_sh.at[pl.ds(s * R0, R0)])
        if rem:
            @pl.when(s == NS - 1)
            def _():
                pltpu.sync_copy(zeros_hbm.at[pl.ds(NS * R0, rem)],
                                accs_sh.at[pl.ds(NS * R0, rem)])
                pltpu.sync_copy(zeros_hbm.at[pl.ds(NS * R0, rem)],
                                accr_sh.at[pl.ds(NS * R0, rem)])
        pltpu.sync_copy(send_hbm.at[pl.ds(wid * NCH, NCH)], sidx_all)
        pltpu.sync_copy(recv_hbm.at[pl.ds(wid * NCH, NCH)], ridx_all)
        pltpu.sync_copy(ones_hbm, ones_v)
        plsc.subcore_barrier()

        def fire(j, carry):
            pltpu.async_copy(ones_v, accs_sh.at[sidx_all.at[j]], sem, add=True)
            pltpu.async_copy(ones_v, accr_sh.at[ridx_all.at[j]], sem, add=True)
            return carry

        lax.fori_loop(0, NCH, fire, 0)

        def drain(j, carry):
            pltpu.make_async_copy(ones_v, accs_sh.at[sidx_all.at[j]], sem).wait()
            pltpu.make_async_copy(ones_v, accr_sh.at[ridx_all.at[j]], sem).wait()
            return carry

        lax.fori_loop(0, NCH, drain, 0)
        plsc.subcore_barrier()
        pltpu.sync_copy(accs_sh.at[pl.ds(s * R0, R0)],
                        outs_hbm.at[pl.ds(c * N + s * R0, R0)])
        pltpu.sync_copy(accr_sh.at[pl.ds(s * R0, R0)],
                        outr_hbm.at[pl.ds(c * N + s * R0, R0)])
        if rem:
            @pl.when(s == NS - 1)
            def _():
                pltpu.sync_copy(accs_sh.at[pl.ds(NS * R0, rem)],
                                outs_hbm.at[pl.ds(c * N + NS * R0, rem)])
                pltpu.sync_copy(accr_sh.at[pl.ds(NS * R0, rem)],
                                outr_hbm.at[pl.ds(c * N + NS * R0, rem)])

    return deg


@functools.lru_cache(maxsize=None)
def _make_conv(N, D, E):
    info = plsc.get_sparse_core_info()
    NC, NS = info.num_cores, info.num_subcores
    NW = NC * NS
    EPW = E // NW
    C = 128
    nfull = EPW // C
    tail = EPW - nfull * C
    R0 = (N // NS) & ~7
    rem = N - NS * R0

    mesh = plsc.VectorSubcoreMesh(core_axis_name="c", subcore_axis_name="s")

    @functools.partial(
        pl.kernel,
        mesh=mesh,
        out_type=jax.ShapeDtypeStruct((NC * N, D), jnp.float32),
        scratch_types=[
            pltpu.VMEM_SHARED((N, D), jnp.float32),
            pltpu.VMEM((C,), jnp.int32),
            pltpu.VMEM((C,), jnp.int32),
            pltpu.VMEM((C, D), jnp.float32),
            pltpu.VMEM((max(tail, 8),), jnp.int32),
            pltpu.VMEM((max(tail, 8),), jnp.int32),
            pltpu.VMEM((max(tail, 8), D), jnp.float32),
            pltpu.SemaphoreType.DMA,
        ],
    )
    def conv(hs_hbm, send_hbm, recv_hbm, zeros_hbm, out_hbm,
             acc_sh, sidx, ridx, rows, sidx_t, ridx_t, rows_t, sem):
        c = lax.axis_index("c")
        s = lax.axis_index("s")
        wid = c * NS + s
        pltpu.sync_copy(zeros_hbm.at[pl.ds(s * R0, R0)],
                        acc_sh.at[pl.ds(s * R0, R0)])
        if rem:
            @pl.when(s == NS - 1)
            def _():
                pltpu.sync_copy(zeros_hbm.at[pl.ds(NS * R0, rem)],
                                acc_sh.at[pl.ds(NS * R0, rem)])
        plsc.subcore_barrier()
        ebase = wid * EPW

        def body(j, carry):
            base = ebase + j * C
            pltpu.sync_copy(send_hbm.at[pl.ds(base, C)], sidx)
            pltpu.sync_copy(recv_hbm.at[pl.ds(base, C)], ridx)
            pltpu.async_copy(hs_hbm.at[sidx], rows, sem).wait()
            pltpu.sync_copy(rows, acc_sh.at[ridx], add=True)
            return carry

        lax.fori_loop(0, nfull, body, 0)
        if tail:
            base = ebase + nfull * C
            pltpu.sync_copy(send_hbm.at[pl.ds(base, tail)], sidx_t)
            pltpu.sync_copy(recv_hbm.at[pl.ds(base, tail)], ridx_t)
            pltpu.async_copy(hs_hbm.at[sidx_t], rows_t, sem).wait()
            pltpu.sync_copy(rows_t, acc_sh.at[ridx_t], add=True)
        plsc.subcore_barrier()
        pltpu.sync_copy(acc_sh.at[pl.ds(s * R0, R0)],
                        out_hbm.at[pl.ds(c * N + s * R0, R0)])
        if rem:
            @pl.when(s == NS - 1)
            def _():
                pltpu.sync_copy(acc_sh.at[pl.ds(NS * R0, rem)],
                                out_hbm.at[pl.ds(c * N + NS * R0, rem)])

    return conv


_STEPS = 2
_ROWS = 2000


def kernel(nodes, senders, receivers, n_node, W_embed, b_embed, W1, b1, W2, b2,
           ln_scale, ln_bias, W_dec, b_dec):
    N, D = nodes.shape
    L = W_embed.shape[1]
    E = senders.shape[0]
    G = n_node.shape[0]
    OG = W_dec.shape[1]

    b_embed2 = b_embed.reshape(1, L)
    b12 = b1.reshape(1, L)
    b22 = b2.reshape(1, L)
    lns2 = ln_scale.reshape(1, L)
    lnb2 = ln_bias.reshape(1, L)
    b_dec2 = b_dec.reshape(1, OG)

    x = _tc_embed(nodes, W_embed, b_embed2, _ROWS)

    NW, NCHT = _edge_layout(E)
    send_deg = _build_edge_chunks(senders, N, NW, NCHT)
    recv_deg = _build_edge_chunks(receivers, N, NW, NCHT)
    zeros_deg = jnp.zeros((N, _DEGW), jnp.float32)
    ones_w = jnp.ones((_C, _DEGW), jnp.float32)
    deg_fn = _make_deg(N, E)
    deg_s, deg_r = deg_fn(send_deg, recv_deg, zeros_deg, ones_w)
    info = plsc.get_sparse_core_info()
    NC = info.num_cores
    deg_s = deg_s.reshape(NC, N, _DEGW)
    deg_r = deg_r.reshape(NC, N, _DEGW)

    zeros_nd = jnp.zeros((N, D), jnp.float32)
    conv_fn = _make_conv(N, D, E)
    for _ in range(_STEPS):
        hs = _tc_mlp(x, W1, b12, W2, b22, deg_s, _ROWS)
        p = conv_fn(hs, senders, receivers, zeros_nd).reshape(NC, N, D)
        x = _tc_update(x, p, hs, deg_r, lns2, lnb2, _ROWS)

    graph_idx = jnp.repeat(jnp.arange(G, dtype=jnp.int32), n_node, axis=0,
                           total_repeat_length=N)
    seg = (graph_idx[None, :] == jnp.arange(G, dtype=jnp.int32)[:, None]
           ).astype(jnp.float32)
    cnt = jnp.maximum(n_node.astype(jnp.float32), 1.0).reshape(G, 1)
    return _tc_pool(x, seg, cnt, W_dec, b_dec2)
